# Initial kernel scaffold; baseline (speedup 1.0000x reference)
import jax
import jax.numpy as jnp
from jax.experimental import pallas as pl


def kernel(feat0, feat1, feat2, feat3, boxes):
    raise NotImplementedError("write your pallas kernel here")


# speedup vs baseline: 20.7523x; 20.7523x over previous
import functools

import jax
import jax.numpy as jnp
from jax import lax
from jax.experimental import pallas as pl
from jax.experimental.pallas import tpu as pltpu
from jax.experimental.pallas import tpu_sc as plsc

_B = 2
_NB = 256
_NROI = _B * _NB
_C = 256
_PH = _PW = 7
_SR = 2
_SCALES = (0.25, 0.125, 0.0625, 0.03125)
_HS = (256, 128, 64, 32)
_CANON_SCALE = 224.0
_CANON_LVL = 4.0
_BASES = (0, 131072, 163840, 172032)
_TABLE_ROWS = 174080
_NOUT = _NROI * _PH * _PW
_NCON = 16

_NC, _NS, _L = 2, 16, 16
_NW = _NC * _NS
_ROWS_PER_W = _NOUT // _NW
_CHUNK = 8
_STEPS = _ROWS_PER_W // _CHUNK


def _prep(boxes):
    flat = boxes.reshape(-1, 4)
    bx1, by1, bx2, by2 = flat[:, 0], flat[:, 1], flat[:, 2], flat[:, 3]
    area = (bx2 - bx1) * (by2 - by1)
    s = jnp.sqrt(area)
    lvl = jnp.floor(_CANON_LVL + jnp.log2(s / _CANON_SCALE) + 1e-6)
    lvl = (jnp.clip(lvl, 2.0, 5.0) - 2.0).astype(jnp.int32)

    scale = jnp.asarray(_SCALES, jnp.float32)[lvl]
    hsf = jnp.asarray(_HS, jnp.float32)[lvl]
    hsi = jnp.asarray(_HS, jnp.int32)[lvl]
    base = jnp.asarray(_BASES, jnp.int32)[lvl]
    hw = (hsi * hsi).astype(jnp.int32)
    bidx = jnp.repeat(jnp.arange(_B, dtype=jnp.int32), _NB)

    x1 = bx1 * scale
    y1 = by1 * scale
    x2 = bx2 * scale
    y2 = by2 * scale
    roi_w = jnp.maximum(x2 - x1, 1.0)
    roi_h = jnp.maximum(y2 - y1, 1.0)
    bin_w = roi_w / _PW
    bin_h = roi_h / _PH

    sub = (jnp.arange(_SR, dtype=jnp.float32) + 0.5) / _SR
    g = (jnp.arange(_PH, dtype=jnp.float32)[:, None] + sub[None, :]).reshape(-1)

    def axis_terms(lo, binsz):
        pts = lo[:, None] + g[None, :] * binsz[:, None]
        valid = (pts >= -1.0) & (pts <= hsf[:, None])
        ptc = jnp.maximum(pts, 0.0)
        i0 = jnp.clip(jnp.floor(ptc), 0.0, hsf[:, None] - 1.0).astype(jnp.int32)
        i1 = jnp.minimum(i0 + 1, hsi[:, None] - 1)
        frac = jnp.where(i0 < hsi[:, None] - 1, ptc - i0.astype(jnp.float32), 0.0)
        corn = jnp.stack([i0, i1], axis=-1)
        wc = jnp.stack([1.0 - frac, frac], axis=-1)
        return corn, wc, valid

    ycorn, wy, vy = axis_terms(y1, bin_h)
    xcorn, wx, vx = axis_terms(x1, bin_w)

    ycorn = ycorn.reshape(_NROI, _PH, 1, _SR, 1, 2, 1)
    wy = wy.reshape(_NROI, _PH, 1, _SR, 1, 2, 1)
    vy = vy.reshape(_NROI, _PH, 1, _SR, 1, 1, 1)
    xcorn = xcorn.reshape(_NROI, 1, _PW, 1, _SR, 1, 2)
    wx = wx.reshape(_NROI, 1, _PW, 1, _SR, 1, 2)
    vx = vx.reshape(_NROI, 1, _PW, 1, _SR, 1, 1)

    off = (base + bidx * hw).reshape(_NROI, 1, 1, 1, 1, 1, 1)
    stride = hsi.reshape(_NROI, 1, 1, 1, 1, 1, 1)
    idx = off + ycorn * stride + xcorn
    idx = jnp.broadcast_to(idx, (_NROI, _PH, _PW, _SR, _SR, 2, 2))
    idx = idx.reshape(_NOUT * _NCON)

    wt = wy * wx * (vy & vx).astype(jnp.float32) * (1.0 / (_SR * _SR))
    wt = jnp.broadcast_to(wt, (_NROI, _PH, _PW, _SR, _SR, 2, 2))
    wt = wt.reshape(_NOUT, _NCON)
    wts_exp = jnp.broadcast_to(wt[:, :, None], (_NOUT, _NCON, _L))
    wts_exp = wts_exp.reshape(_NOUT, _NCON * _L)
    return idx, wts_exp


def _sc_gather_reduce(table, idx, wts):
    mesh = plsc.VectorSubcoreMesh(core_axis_name="c", subcore_axis_name="s")

    @functools.partial(
        pl.kernel,
        out_type=jax.ShapeDtypeStruct((_NOUT, _C), jnp.float32),
        mesh=mesh,
        scratch_types=[
            pltpu.VMEM((_CHUNK * _NCON,), jnp.int32),
            pltpu.VMEM((_CHUNK * _NCON, _C), jnp.float32),
            pltpu.VMEM((_CHUNK, _C), jnp.float32),
            pltpu.VMEM((_CHUNK, _C), jnp.float32),
            pltpu.SemaphoreType.DMA,
        ],
    )
    def k(table_hbm, idx_hbm, wts_hbm, out_hbm, idx_v, rows_v, wts_v, out_v, sem):
        wid = lax.axis_index("s") * _NC + lax.axis_index("c")
        row0 = wid * _ROWS_PER_W

        @pl.loop(0, _STEPS)
        def _(st):
            r0 = row0 + st * _CHUNK
            pltpu.sync_copy(idx_hbm.at[pl.ds(r0 * _NCON, _CHUNK * _NCON)], idx_v)
            pltpu.async_copy(table_hbm.at[idx_v], rows_v, sem).wait()
            pltpu.sync_copy(wts_hbm.at[pl.ds(r0, _CHUNK)], wts_v)

            @pl.loop(0, _CHUNK)
            def _(r):
                wvec = [wts_v[r, pl.ds(j * _L, _L)] for j in range(_NCON)]
                for c in range(_C // _L):
                    acc = wvec[0] * rows_v[r * _NCON, pl.ds(c * _L, _L)]
                    for j in range(1, _NCON):
                        acc = acc + wvec[j] * rows_v[r * _NCON + j, pl.ds(c * _L, _L)]
                    out_v[r, pl.ds(c * _L, _L)] = acc

            pltpu.sync_copy(out_v, out_hbm.at[pl.ds(r0, _CHUNK)])

    return k(table, idx, wts)


@jax.jit
def kernel(feat0, feat1, feat2, feat3, boxes):
    table = jnp.concatenate(
        [jnp.transpose(f, (0, 2, 3, 1)).reshape(-1, _C)
         for f in (feat0, feat1, feat2, feat3)],
        axis=0,
    )
    idx, wts = _prep(boxes)
    out_rows = _sc_gather_reduce(table, idx, wts)
    return jnp.transpose(out_rows.reshape(_NROI, _PH, _PW, _C), (0, 3, 1, 2))


# speedup vs baseline: 27.8608x; 1.3425x over previous
import functools

import jax
import jax.numpy as jnp
from jax import lax
from jax.experimental import pallas as pl
from jax.experimental.pallas import tpu as pltpu
from jax.experimental.pallas import tpu_sc as plsc

_B = 2
_NB = 256
_NROI = _B * _NB
_C = 256
_PH = _PW = 7
_SR = 2
_SCALES = (0.25, 0.125, 0.0625, 0.03125)
_HS = (256, 128, 64, 32)
_CANON_SCALE = 224.0
_CANON_LVL = 4.0
_BASES = (0, 131072, 163840, 172032)
_TABLE_ROWS = 174080
_NOUT = _NROI * _PH * _PW
_NCON = 16

_NC, _NS, _L = 2, 16, 16
_NW = _NC * _NS
_ROWS_PER_W = _NOUT // _NW
_CHUNK = 8
_STEPS = _ROWS_PER_W // _CHUNK


def _prep(boxes):
    flat = boxes.reshape(-1, 4)
    bx1, by1, bx2, by2 = flat[:, 0], flat[:, 1], flat[:, 2], flat[:, 3]
    area = (bx2 - bx1) * (by2 - by1)
    s = jnp.sqrt(area)
    lvl = jnp.floor(_CANON_LVL + jnp.log2(s / _CANON_SCALE) + 1e-6)
    lvl = (jnp.clip(lvl, 2.0, 5.0) - 2.0).astype(jnp.int32)

    scale = jnp.asarray(_SCALES, jnp.float32)[lvl]
    hsf = jnp.asarray(_HS, jnp.float32)[lvl]
    hsi = jnp.asarray(_HS, jnp.int32)[lvl]
    base = jnp.asarray(_BASES, jnp.int32)[lvl]
    hw = (hsi * hsi).astype(jnp.int32)
    bidx = jnp.repeat(jnp.arange(_B, dtype=jnp.int32), _NB)

    x1 = bx1 * scale
    y1 = by1 * scale
    x2 = bx2 * scale
    y2 = by2 * scale
    roi_w = jnp.maximum(x2 - x1, 1.0)
    roi_h = jnp.maximum(y2 - y1, 1.0)
    bin_w = roi_w / _PW
    bin_h = roi_h / _PH

    sub = (jnp.arange(_SR, dtype=jnp.float32) + 0.5) / _SR
    g = (jnp.arange(_PH, dtype=jnp.float32)[:, None] + sub[None, :]).reshape(-1)

    def axis_terms(lo, binsz):
        pts = lo[:, None] + g[None, :] * binsz[:, None]
        valid = (pts >= -1.0) & (pts <= hsf[:, None])
        ptc = jnp.maximum(pts, 0.0)
        i0 = jnp.clip(jnp.floor(ptc), 0.0, hsf[:, None] - 1.0).astype(jnp.int32)
        i1 = jnp.minimum(i0 + 1, hsi[:, None] - 1)
        frac = jnp.where(i0 < hsi[:, None] - 1, ptc - i0.astype(jnp.float32), 0.0)
        corn = jnp.stack([i0, i1], axis=-1)
        wc = jnp.stack([1.0 - frac, frac], axis=-1)
        return corn, wc, valid

    ycorn, wy, vy = axis_terms(y1, bin_h)
    xcorn, wx, vx = axis_terms(x1, bin_w)

    ycorn = ycorn.reshape(_NROI, _PH, 1, _SR, 1, 2, 1)
    wy = wy.reshape(_NROI, _PH, 1, _SR, 1, 2, 1)
    vy = vy.reshape(_NROI, _PH, 1, _SR, 1, 1, 1)
    xcorn = xcorn.reshape(_NROI, 1, _PW, 1, _SR, 1, 2)
    wx = wx.reshape(_NROI, 1, _PW, 1, _SR, 1, 2)
    vx = vx.reshape(_NROI, 1, _PW, 1, _SR, 1, 1)

    off = (base + bidx * hw).reshape(_NROI, 1, 1, 1, 1, 1, 1)
    stride = hsi.reshape(_NROI, 1, 1, 1, 1, 1, 1)
    idx = off + ycorn * stride + xcorn
    idx = jnp.broadcast_to(idx, (_NROI, _PH, _PW, _SR, _SR, 2, 2))
    idx = idx.reshape(_NOUT * _NCON)

    wt = wy * wx * (vy & vx).astype(jnp.float32) * (1.0 / (_SR * _SR))
    wt = jnp.broadcast_to(wt, (_NROI, _PH, _PW, _SR, _SR, 2, 2))
    wt = wt.reshape(_NOUT, _NCON)
    wts_exp = jnp.broadcast_to(wt[:, :, None], (_NOUT, _NCON, _L))
    wts_exp = wts_exp.reshape(_NOUT, _NCON * _L)
    return idx, wts_exp


def _sc_gather_reduce(table, idx, wts):
    mesh = plsc.VectorSubcoreMesh(core_axis_name="c", subcore_axis_name="s")
    g_per_chunk = _CHUNK * _NCON

    @functools.partial(
        pl.kernel,
        out_type=jax.ShapeDtypeStruct((_NOUT, _C), jnp.float32),
        mesh=mesh,
        scratch_types=[
            pltpu.VMEM((_ROWS_PER_W * _NCON,), jnp.int32),
            [pltpu.VMEM((g_per_chunk, _C), jnp.float32)] * 2,
            [pltpu.VMEM((_CHUNK, _C), jnp.float32)] * 2,
            [pltpu.VMEM((_CHUNK, _C), jnp.float32)] * 2,
            [pltpu.SemaphoreType.DMA] * 2,
            [pltpu.SemaphoreType.DMA] * 2,
            [pltpu.SemaphoreType.DMA] * 2,
        ],
    )
    def k(table_hbm, idx_hbm, wts_hbm, out_hbm,
          idx_v, rows_v, wts_v, out_v, gsem, wsem, osem):
        wid = lax.axis_index("s") * _NC + lax.axis_index("c")
        row0 = wid * _ROWS_PER_W
        pltpu.sync_copy(idx_hbm.at[pl.ds(row0 * _NCON, _ROWS_PER_W * _NCON)],
                        idx_v)

        def start(c, buf):
            pltpu.async_copy(
                table_hbm.at[idx_v.at[pl.ds(c * g_per_chunk, g_per_chunk)]],
                rows_v[buf], gsem[buf])
            pltpu.async_copy(wts_hbm.at[pl.ds(row0 + c * _CHUNK, _CHUNK)],
                             wts_v[buf], wsem[buf])

        def compute_store(c, buf, first):
            pltpu.make_async_copy(
                table_hbm.at[idx_v.at[pl.ds(0, g_per_chunk)]],
                rows_v[buf], gsem[buf]).wait()
            pltpu.make_async_copy(wts_hbm.at[pl.ds(0, _CHUNK)],
                                  wts_v[buf], wsem[buf]).wait()

            @pl.when(jnp.logical_not(first))
            def _():
                pltpu.make_async_copy(out_v[buf],
                                      out_hbm.at[pl.ds(0, _CHUNK)],
                                      osem[buf]).wait()

            @pl.loop(0, _CHUNK)
            def _(r):
                wvec = [wts_v[buf][r, pl.ds(j * _L, _L)] for j in range(_NCON)]
                for cc in range(_C // _L):
                    acc = wvec[0] * rows_v[buf][r * _NCON, pl.ds(cc * _L, _L)]
                    for j in range(1, _NCON):
                        acc = acc + wvec[j] * rows_v[buf][r * _NCON + j,
                                                         pl.ds(cc * _L, _L)]
                    out_v[buf][r, pl.ds(cc * _L, _L)] = acc

            pltpu.async_copy(out_v[buf],
                             out_hbm.at[pl.ds(row0 + c * _CHUNK, _CHUNK)],
                             osem[buf])

        start(0, 0)

        @pl.loop(0, _STEPS // 2)
        def _(kk):
            ca = 2 * kk
            cb = ca + 1
            start(cb, 1)
            compute_store(ca, 0, kk == 0)
            start(jnp.minimum(ca + 2, _STEPS - 1), 0)
            compute_store(cb, 1, kk == 0)

        pltpu.make_async_copy(
            table_hbm.at[idx_v.at[pl.ds(0, g_per_chunk)]],
            rows_v[0], gsem[0]).wait()
        pltpu.make_async_copy(wts_hbm.at[pl.ds(0, _CHUNK)],
                              wts_v[0], wsem[0]).wait()
        pltpu.make_async_copy(out_v[0], out_hbm.at[pl.ds(0, _CHUNK)],
                              osem[0]).wait()
        pltpu.make_async_copy(out_v[1], out_hbm.at[pl.ds(0, _CHUNK)],
                              osem[1]).wait()

    return k(table, idx, wts)


@jax.jit
def kernel(feat0, feat1, feat2, feat3, boxes):
    table = jnp.concatenate(
        [jnp.transpose(f, (0, 2, 3, 1)).reshape(-1, _C)
         for f in (feat0, feat1, feat2, feat3)],
        axis=0,
    )
    idx, wts = _prep(boxes)
    out_rows = _sc_gather_reduce(table, idx, wts)
    return jnp.transpose(out_rows.reshape(_NROI, _PH, _PW, _C), (0, 3, 1, 2))


# speedup vs baseline: 40.3851x; 1.4495x over previous
import functools

import jax
import jax.numpy as jnp
from jax import lax
from jax.experimental import pallas as pl
from jax.experimental.pallas import tpu as pltpu
from jax.experimental.pallas import tpu_sc as plsc

_B = 2
_NB = 256
_NROI = _B * _NB
_C = 256
_PH = _PW = 7
_SR = 2
_SCALES = (0.25, 0.125, 0.0625, 0.03125)
_HS = (256, 128, 64, 32)
_CANON_SCALE = 224.0
_CANON_LVL = 4.0
_BASES = (0, 131072, 163840, 172032)
_TABLE_ROWS = 174080
_NOUT = _NROI * _PH * _PW
_NCON = 16

_NC, _NS, _L = 2, 16, 16
_NW = _NC * _NS
_ROWS_PER_W = _NOUT // _NW
_CHUNK = 8
_STEPS = _ROWS_PER_W // _CHUNK


def _sel_mats():
    import numpy as np
    q = np.arange(784)
    pq, j = q // 16, q % 16
    py, px = pq // 7, pq % 7
    sy, sx = j // 8, (j // 4) % 2
    cy, cx = (j // 2) % 2, j % 2
    ay = cy * 14 + (py * 2 + sy)
    ax = cx * 14 + (px * 2 + sx)
    sel_y = np.zeros((28, 784), np.float32)
    sel_x = np.zeros((28, 784), np.float32)
    sel_y[ay, q] = 1.0
    sel_x[ax, q] = 1.0
    rep = np.zeros((16, 256), np.float32)
    rep[np.arange(256) // 16, np.arange(256)] = 1.0
    return sel_y, sel_x, rep


_SEL_Y, _SEL_X, _REP = _sel_mats()


def _prep(boxes):
    flat = boxes.reshape(-1, 4)
    bx1, by1, bx2, by2 = flat[:, 0], flat[:, 1], flat[:, 2], flat[:, 3]
    area = (bx2 - bx1) * (by2 - by1)
    s = jnp.sqrt(area)
    lvl = jnp.floor(_CANON_LVL + jnp.log2(s / _CANON_SCALE) + 1e-6)
    lvl = (jnp.clip(lvl, 2.0, 5.0) - 2.0).astype(jnp.int32)

    scale = jnp.asarray(_SCALES, jnp.float32)[lvl]
    hsf = jnp.asarray(_HS, jnp.float32)[lvl]
    hsi = jnp.asarray(_HS, jnp.int32)[lvl]
    base = jnp.asarray(_BASES, jnp.int32)[lvl]
    hw = (hsi * hsi).astype(jnp.int32)
    bidx = jnp.repeat(jnp.arange(_B, dtype=jnp.int32), _NB)

    x1 = bx1 * scale
    y1 = by1 * scale
    x2 = bx2 * scale
    y2 = by2 * scale
    roi_w = jnp.maximum(x2 - x1, 1.0)
    roi_h = jnp.maximum(y2 - y1, 1.0)
    bin_w = roi_w / _PW
    bin_h = roi_h / _PH

    sub = (jnp.arange(_SR, dtype=jnp.float32) + 0.5) / _SR
    g = (jnp.arange(_PH, dtype=jnp.float32)[:, None] + sub[None, :]).reshape(-1)

    def axis_terms(lo, binsz):
        pts = lo[:, None] + g[None, :] * binsz[:, None]
        valid = (pts >= -1.0) & (pts <= hsf[:, None])
        ptc = jnp.maximum(pts, 0.0)
        i0 = jnp.clip(jnp.floor(ptc), 0.0, hsf[:, None] - 1.0)
        i1 = jnp.minimum(i0 + 1.0, hsf[:, None] - 1.0)
        frac = jnp.where(i0 < hsf[:, None] - 1.0, ptc - i0, 0.0)
        vf = valid.astype(jnp.float32)
        corn = jnp.concatenate([i0, i1], axis=1)
        wc = jnp.concatenate([(1.0 - frac) * vf, frac * vf], axis=1)
        return corn, wc

    ycorn, wy = axis_terms(y1, bin_h)
    xcorn, wx = axis_terms(x1, bin_w)

    hp = jax.lax.Precision.HIGHEST
    sel_y = jnp.asarray(_SEL_Y)
    sel_x = jnp.asarray(_SEL_X)
    ysel = jnp.dot(ycorn, sel_y, precision=hp)
    xsel = jnp.dot(xcorn, sel_x, precision=hp)
    wysel = jnp.dot(wy, sel_y, precision=hp)
    wxsel = jnp.dot(wx, sel_x, precision=hp)

    off = (base + bidx * hw).astype(jnp.float32)[:, None]
    stride = hsf[:, None]
    idx = jnp.round(off + ysel * stride + xsel).astype(jnp.int32)
    idx = idx.reshape(_NOUT * _NCON)

    wt = (wysel * wxsel * (1.0 / (_SR * _SR))).reshape(_NOUT, _NCON)
    wts_exp = jnp.dot(wt, jnp.asarray(_REP), precision=hp)
    return idx, wts_exp


def _sc_gather_reduce(table, idx, wts):
    mesh = plsc.VectorSubcoreMesh(core_axis_name="c", subcore_axis_name="s")
    g_per_chunk = _CHUNK * _NCON

    @functools.partial(
        pl.kernel,
        out_type=jax.ShapeDtypeStruct((_NOUT, _C), jnp.float32),
        mesh=mesh,
        scratch_types=[
            pltpu.VMEM((_ROWS_PER_W * _NCON,), jnp.int32),
            [pltpu.VMEM((g_per_chunk, _C), jnp.float32)] * 2,
            [pltpu.VMEM((_CHUNK, _C), jnp.float32)] * 2,
            [pltpu.VMEM((_CHUNK, _C), jnp.float32)] * 2,
            [pltpu.SemaphoreType.DMA] * 2,
            [pltpu.SemaphoreType.DMA] * 2,
            [pltpu.SemaphoreType.DMA] * 2,
        ],
    )
    def k(table_hbm, idx_hbm, wts_hbm, out_hbm,
          idx_v, rows_v, wts_v, out_v, gsem, wsem, osem):
        wid = lax.axis_index("s") * _NC + lax.axis_index("c")
        row0 = wid * _ROWS_PER_W
        pltpu.sync_copy(idx_hbm.at[pl.ds(row0 * _NCON, _ROWS_PER_W * _NCON)],
                        idx_v)

        def start(c, buf):
            pltpu.async_copy(
                table_hbm.at[idx_v.at[pl.ds(c * g_per_chunk, g_per_chunk)]],
                rows_v[buf], gsem[buf])
            pltpu.async_copy(wts_hbm.at[pl.ds(row0 + c * _CHUNK, _CHUNK)],
                             wts_v[buf], wsem[buf])

        def compute_store(c, buf, first):
            pltpu.make_async_copy(
                table_hbm.at[idx_v.at[pl.ds(0, g_per_chunk)]],
                rows_v[buf], gsem[buf]).wait()
            pltpu.make_async_copy(wts_hbm.at[pl.ds(0, _CHUNK)],
                                  wts_v[buf], wsem[buf]).wait()

            @pl.when(jnp.logical_not(first))
            def _():
                pltpu.make_async_copy(out_v[buf],
                                      out_hbm.at[pl.ds(0, _CHUNK)],
                                      osem[buf]).wait()

            @pl.loop(0, _CHUNK)
            def _(r):
                wvec = [wts_v[buf][r, pl.ds(j * _L, _L)] for j in range(_NCON)]
                for cc in range(_C // _L):
                    acc = wvec[0] * rows_v[buf][r * _NCON, pl.ds(cc * _L, _L)]
                    for j in range(1, _NCON):
                        acc = acc + wvec[j] * rows_v[buf][r * _NCON + j,
                                                         pl.ds(cc * _L, _L)]
                    out_v[buf][r, pl.ds(cc * _L, _L)] = acc

            pltpu.async_copy(out_v[buf],
                             out_hbm.at[pl.ds(row0 + c * _CHUNK, _CHUNK)],
                             osem[buf])

        start(0, 0)

        @pl.loop(0, _STEPS // 2)
        def _(kk):
            ca = 2 * kk
            cb = ca + 1
            start(cb, 1)
            compute_store(ca, 0, kk == 0)
            start(jnp.minimum(ca + 2, _STEPS - 1), 0)
            compute_store(cb, 1, kk == 0)

        pltpu.make_async_copy(
            table_hbm.at[idx_v.at[pl.ds(0, g_per_chunk)]],
            rows_v[0], gsem[0]).wait()
        pltpu.make_async_copy(wts_hbm.at[pl.ds(0, _CHUNK)],
                              wts_v[0], wsem[0]).wait()
        pltpu.make_async_copy(out_v[0], out_hbm.at[pl.ds(0, _CHUNK)],
                              osem[0]).wait()
        pltpu.make_async_copy(out_v[1], out_hbm.at[pl.ds(0, _CHUNK)],
                              osem[1]).wait()

    return k(table, idx, wts)


@jax.jit
def kernel(feat0, feat1, feat2, feat3, boxes):
    table = jnp.concatenate(
        [jnp.transpose(f, (0, 2, 3, 1)).reshape(-1, _C)
         for f in (feat0, feat1, feat2, feat3)],
        axis=0,
    )
    idx, wts = _prep(boxes)
    out_rows = _sc_gather_reduce(table, idx, wts)
    return jnp.transpose(out_rows.reshape(_NROI, _PH, _PW, _C), (0, 3, 1, 2))


# speedup vs baseline: 46.6639x; 1.1555x over previous
import dataclasses
import functools

import jax
import jax.numpy as jnp
from jax import lax
from jax.experimental import pallas as pl
from jax.experimental.pallas import tpu as pltpu
from jax.experimental.pallas import tpu_sc as plsc

_B = 2
_NB = 256
_NROI = _B * _NB
_C = 256
_PH = _PW = 7
_SR = 2
_SCALES = (0.25, 0.125, 0.0625, 0.03125)
_HS = (256, 128, 64, 32)
_CANON_SCALE = 224.0
_CANON_LVL = 4.0
_BASES = (0, 131072, 163840, 172032)
_TABLE_ROWS = 174080
_NOUT = _NROI * _PH * _PW
_NCON = 16

_NC, _NS, _L = 2, 16, 16
_NW = _NC * _NS
_ROWS_PER_W = _NOUT // _NW
_CHUNK = 7
_STEPS = _ROWS_PER_W // _CHUNK


def _sel_mats():
    import numpy as np
    q = np.arange(784)
    pq, j = q // 16, q % 16
    py, px = pq // 7, pq % 7
    sy, sx = j // 8, (j // 4) % 2
    cy, cx = (j // 2) % 2, j % 2
    ay = cy * 14 + (py * 2 + sy)
    ax = cx * 14 + (px * 2 + sx)
    sel_y = np.zeros((28, 784), np.float32)
    sel_x = np.zeros((28, 784), np.float32)
    sel_y[ay, q] = 1.0
    sel_x[ax, q] = 1.0
    rep = np.zeros((16, 256), np.float32)
    rep[np.arange(256) // 16, np.arange(256)] = 1.0
    return sel_y, sel_x, rep


_SEL_Y, _SEL_X, _REP = _sel_mats()


def _prep(boxes):
    flat = boxes.reshape(-1, 4)
    bx1, by1, bx2, by2 = flat[:, 0], flat[:, 1], flat[:, 2], flat[:, 3]
    area = (bx2 - bx1) * (by2 - by1)
    s = jnp.sqrt(area)
    lvl = jnp.floor(_CANON_LVL + jnp.log2(s / _CANON_SCALE) + 1e-6)
    lvl = (jnp.clip(lvl, 2.0, 5.0) - 2.0).astype(jnp.int32)

    scale = jnp.asarray(_SCALES, jnp.float32)[lvl]
    hsf = jnp.asarray(_HS, jnp.float32)[lvl]
    hsi = jnp.asarray(_HS, jnp.int32)[lvl]
    base = jnp.asarray(_BASES, jnp.int32)[lvl]
    hw = (hsi * hsi).astype(jnp.int32)
    bidx = jnp.repeat(jnp.arange(_B, dtype=jnp.int32), _NB)

    x1 = bx1 * scale
    y1 = by1 * scale
    x2 = bx2 * scale
    y2 = by2 * scale
    roi_w = jnp.maximum(x2 - x1, 1.0)
    roi_h = jnp.maximum(y2 - y1, 1.0)
    bin_w = roi_w / _PW
    bin_h = roi_h / _PH

    sub = (jnp.arange(_SR, dtype=jnp.float32) + 0.5) / _SR
    g = (jnp.arange(_PH, dtype=jnp.float32)[:, None] + sub[None, :]).reshape(-1)

    def axis_terms(lo, binsz):
        pts = lo[:, None] + g[None, :] * binsz[:, None]
        valid = (pts >= -1.0) & (pts <= hsf[:, None])
        ptc = jnp.maximum(pts, 0.0)
        i0 = jnp.clip(jnp.floor(ptc), 0.0, hsf[:, None] - 1.0)
        i1 = jnp.minimum(i0 + 1.0, hsf[:, None] - 1.0)
        frac = jnp.where(i0 < hsf[:, None] - 1.0, ptc - i0, 0.0)
        vf = valid.astype(jnp.float32)
        corn = jnp.concatenate([i0, i1], axis=1)
        wc = jnp.concatenate([(1.0 - frac) * vf, frac * vf], axis=1)
        return corn, wc

    ycorn, wy = axis_terms(y1, bin_h)
    xcorn, wx = axis_terms(x1, bin_w)

    hp = jax.lax.Precision.HIGHEST
    sel_y = jnp.asarray(_SEL_Y)
    sel_x = jnp.asarray(_SEL_X)
    ysel = jnp.dot(ycorn, sel_y, precision=hp)
    xsel = jnp.dot(xcorn, sel_x, precision=hp)
    wysel = jnp.dot(wy, sel_y, precision=hp)
    wxsel = jnp.dot(wx, sel_x, precision=hp)

    off = (bidx * hw).astype(jnp.float32)[:, None]
    stride = hsf[:, None]
    idx = jnp.round(off + ysel * stride + xsel).astype(jnp.int32)
    idx = idx.reshape(_NOUT * _NCON)

    wt = (wysel * wxsel * (1.0 / (_SR * _SR))).reshape(_NOUT, _NCON)
    wts_exp = jnp.dot(wt, jnp.asarray(_REP), precision=hp)
    lvl_chunk = jnp.repeat(lvl, _PH * _L)
    return idx, wts_exp, lvl_chunk


def _sc_gather_reduce(tables, idx, wts, lvlc):
    mesh = plsc.VectorSubcoreMesh(core_axis_name="c", subcore_axis_name="s")
    g_per_chunk = _CHUNK * _NCON

    ch_elems = _CHUNK * _C

    cp = pltpu.CompilerParams()
    if "needs_layout_passes" in pltpu.CompilerParams.__dataclass_fields__:
        cp = dataclasses.replace(cp, needs_layout_passes=False)

    @functools.partial(
        pl.kernel,
        compiler_params=cp,
        out_type=jax.ShapeDtypeStruct((_NOUT * _C,), jnp.float32),
        mesh=mesh,
        scratch_types=[
            pltpu.VMEM((_ROWS_PER_W * _NCON,), jnp.int32),
            pltpu.VMEM((_STEPS * _L,), jnp.int32),
            [pltpu.VMEM((g_per_chunk, _C), jnp.float32)] * 2,
            [pltpu.VMEM((ch_elems,), jnp.float32)] * 2,
            [pltpu.VMEM((ch_elems,), jnp.float32)] * 2,
            [pltpu.SemaphoreType.DMA] * 2,
            [pltpu.SemaphoreType.DMA] * 2,
            [pltpu.SemaphoreType.DMA] * 2,
        ],
    )
    def k(t0_hbm, t1_hbm, t2_hbm, t3_hbm, idx_hbm, wts_hbm, lvl_hbm, out_hbm,
          idx_v, lvl_v, rows_v, wts_v, out_v, gsem, wsem, osem):
        wid = lax.axis_index("s") * _NC + lax.axis_index("c")
        row0 = wid * _ROWS_PER_W
        pltpu.sync_copy(idx_hbm.at[pl.ds(row0 * _NCON, _ROWS_PER_W * _NCON)],
                        idx_v)
        pltpu.sync_copy(lvl_hbm.at[pl.ds(wid * _STEPS * _L, _STEPS * _L)],
                        lvl_v)

        def start(c, buf):
            lv = jnp.max(lvl_v[pl.ds(c * _L, _L)])
            sl = idx_v.at[pl.ds(c * g_per_chunk, g_per_chunk)]
            for l, t in enumerate((t0_hbm, t1_hbm, t2_hbm, t3_hbm)):
                @pl.when(lv == l)
                def _():
                    pltpu.async_copy(t.at[sl], rows_v[buf], gsem[buf])
            pltpu.async_copy(
                wts_hbm.at[pl.ds(row0 * _C + c * ch_elems, ch_elems)],
                wts_v[buf], wsem[buf])

        def compute_store(c, buf, first):
            pltpu.make_async_copy(
                t0_hbm.at[idx_v.at[pl.ds(0, g_per_chunk)]],
                rows_v[buf], gsem[buf]).wait()
            pltpu.make_async_copy(wts_hbm.at[pl.ds(0, ch_elems)],
                                  wts_v[buf], wsem[buf]).wait()

            @pl.when(jnp.logical_not(first))
            def _():
                pltpu.make_async_copy(out_v[buf],
                                      out_hbm.at[pl.ds(0, ch_elems)],
                                      osem[buf]).wait()

            @pl.loop(0, _CHUNK)
            def _(r):
                wvec = [wts_v[buf][pl.ds(r * _C + j * _L, _L)]
                        for j in range(_NCON)]
                for cc in range(_C // _L):
                    acc = wvec[0] * rows_v[buf][r * _NCON, pl.ds(cc * _L, _L)]
                    for j in range(1, _NCON):
                        acc = acc + wvec[j] * rows_v[buf][r * _NCON + j,
                                                         pl.ds(cc * _L, _L)]
                    out_v[buf][pl.ds(r * _C + cc * _L, _L)] = acc

            pltpu.async_copy(
                out_v[buf],
                out_hbm.at[pl.ds(row0 * _C + c * ch_elems, ch_elems)],
                osem[buf])

        start(0, 0)

        @pl.loop(0, _STEPS // 2)
        def _(kk):
            ca = 2 * kk
            cb = ca + 1
            start(cb, 1)
            compute_store(ca, 0, kk == 0)
            start(jnp.minimum(ca + 2, _STEPS - 1), 0)
            compute_store(cb, 1, kk == 0)

        pltpu.make_async_copy(
            t0_hbm.at[idx_v.at[pl.ds(0, g_per_chunk)]],
            rows_v[0], gsem[0]).wait()
        pltpu.make_async_copy(wts_hbm.at[pl.ds(0, ch_elems)],
                              wts_v[0], wsem[0]).wait()
        pltpu.make_async_copy(out_v[0], out_hbm.at[pl.ds(0, ch_elems)],
                              osem[0]).wait()
        pltpu.make_async_copy(out_v[1], out_hbm.at[pl.ds(0, ch_elems)],
                              osem[1]).wait()

    return k(*tables, idx, wts.reshape(-1), lvlc)


@jax.jit
def kernel(feat0, feat1, feat2, feat3, boxes):
    tables = [jnp.transpose(f, (0, 2, 3, 1)).reshape(-1, _C)
              for f in (feat0, feat1, feat2, feat3)]
    idx, wts, lvlc = _prep(boxes)
    out_flat = _sc_gather_reduce(tables, idx, wts, lvlc)
    return jnp.transpose(out_flat.reshape(_NROI, _PH, _PW, _C), (0, 3, 1, 2))
